# baseline (speedup 1.0000x reference)
import jax
import jax.numpy as jnp
from jax import lax
from jax.experimental import pallas as pl
from jax.experimental.pallas import tpu as pltpu
from jax.experimental.pallas import tpu_sc as plsc

B, H, W, P = 4, 512, 512, 50000
MARGIN = 1.0
NPTS = B * P
NC, NS, L = 2, 16, 16
NW = NC * NS
NPAD = 200704
PER_W = NPAD // NW
CHUNK = 128
NCHUNK = PER_W // CHUNK
SUB = CHUNK // L

TC_R, TC_C = 784, 256


def _sc_gather(img, ya, xa, yb, xb):
    mesh = plsc.VectorSubcoreMesh(
        core_axis_name="c", subcore_axis_name="s", num_cores=NC, num_subcores=NS
    )

    def body(img_hbm, ya_hbm, xa_hbm, yb_hbm, xb_hbm, za_hbm, zb_hbm,
             ya_v, xa_v, yb_v, xb_v, idxa_v, idxb_v, za_v, zb_v, sem):
        wid = lax.axis_index("s") * NC + lax.axis_index("c")
        base = wid * PER_W

        cps = [
            pltpu.async_copy(src.at[pl.ds(base, PER_W)], dst, sem)
            for src, dst in ((ya_hbm, ya_v), (xa_hbm, xa_v),
                             (yb_hbm, yb_v), (xb_hbm, xb_v))
        ]
        for cp in cps:
            cp.wait()

        def chunk_body(c, carry):
            for u in range(SUB):
                off = c * CHUNK + u * L
                ii = base + off + lax.iota(jnp.int32, L)
                zero = ii * 0
                boff = (jnp.where(ii >= P, H * W, zero)
                        + jnp.where(ii >= 2 * P, H * W, zero)
                        + jnp.where(ii >= 3 * P, H * W, zero))
                idxa_v[pl.ds(off, L)] = boff + ya_v[pl.ds(off, L)] * W + xa_v[pl.ds(off, L)]
                idxb_v[pl.ds(off, L)] = boff + yb_v[pl.ds(off, L)] * W + xb_v[pl.ds(off, L)]
            co = c * CHUNK
            pltpu.async_copy(img_hbm.at[idxa_v.at[pl.ds(co, CHUNK)]],
                             za_v.at[pl.ds(co, CHUNK)], sem)
            pltpu.async_copy(img_hbm.at[idxb_v.at[pl.ds(co, CHUNK)]],
                             zb_v.at[pl.ds(co, CHUNK)], sem)
            return carry

        lax.fori_loop(0, NCHUNK, chunk_body, 0)

        pltpu.make_async_copy(img_hbm.at[pl.ds(0, PER_W)], za_v, sem).wait()
        pltpu.make_async_copy(img_hbm.at[pl.ds(0, PER_W)], zb_v, sem).wait()

        pltpu.sync_copy(za_v, za_hbm.at[pl.ds(base, PER_W)])
        pltpu.sync_copy(zb_v, zb_hbm.at[pl.ds(base, PER_W)])

    f = pl.kernel(
        body,
        out_type=(
            jax.ShapeDtypeStruct((NPAD,), jnp.float32),
            jax.ShapeDtypeStruct((NPAD,), jnp.float32),
        ),
        mesh=mesh,
        scratch_types=[
            pltpu.VMEM((PER_W,), jnp.int32),
            pltpu.VMEM((PER_W,), jnp.int32),
            pltpu.VMEM((PER_W,), jnp.int32),
            pltpu.VMEM((PER_W,), jnp.int32),
            pltpu.VMEM((PER_W,), jnp.int32),
            pltpu.VMEM((PER_W,), jnp.int32),
            pltpu.VMEM((PER_W,), jnp.float32),
            pltpu.VMEM((PER_W,), jnp.float32),
            pltpu.SemaphoreType.DMA,
        ],
    )
    return f(img, ya, xa, yb, xb)


def _tc_loss_body(za_ref, zb_ref, o_ref, out_ref):
    d = za_ref[...] - zb_ref[...]
    o = o_ref[...]
    mask = jnp.abs(o)
    t = jnp.minimum(o * d, MARGIN)
    loss = mask * jnp.log(1.0 + jnp.exp(-t)) \
        + (1.0 - mask) * jnp.maximum(d * d, MARGIN * MARGIN)
    pos = (lax.broadcasted_iota(jnp.int32, (TC_R, TC_C), 0) * TC_C
           + lax.broadcasted_iota(jnp.int32, (TC_R, TC_C), 1))
    loss = jnp.where(pos < NPTS, loss, 0.0)
    out_ref[0, 0] = jnp.sum(loss) / NPTS


def _tc_loss(za, zb, o):
    return pl.pallas_call(
        _tc_loss_body,
        out_shape=jax.ShapeDtypeStruct((1, 1), jnp.float32),
        out_specs=pl.BlockSpec(memory_space=pltpu.SMEM),
    )(za, zb, o)


def _pad_flat_i32(a):
    return jnp.concatenate(
        [a.reshape(-1).astype(jnp.int32), jnp.zeros((NPAD - NPTS,), jnp.int32)]
    )


def kernel(input, x_A, y_A, x_B, y_B, ordinal):
    img = input.reshape(B * H * W)
    ya = _pad_flat_i32(y_A)
    xa = _pad_flat_i32(x_A)
    yb = _pad_flat_i32(y_B)
    xb = _pad_flat_i32(x_B)
    o = jnp.concatenate(
        [ordinal.reshape(-1).astype(jnp.float32),
         jnp.zeros((NPAD - NPTS,), jnp.float32)]
    )
    za, zb = _sc_gather(img, ya, xa, yb, xb)
    out = _tc_loss(za.reshape(TC_R, TC_C), zb.reshape(TC_R, TC_C),
                   o.reshape(TC_R, TC_C))
    return out[0, 0]


# speedup vs baseline: 1.1459x; 1.1459x over previous
import jax
import jax.numpy as jnp
from jax import lax
from jax.experimental import pallas as pl
from jax.experimental.pallas import tpu as pltpu
from jax.experimental.pallas import tpu_sc as plsc

B, H, W, P = 4, 512, 512, 50000
MARGIN = 1.0
NPTS = B * P
NC, NS, L = 2, 16, 16
NW = NC * NS
WPB = NW // B
PPAD = 50176
NPAD = B * PPAD
PER_W = PPAD // WPB
CHUNK = 128
NCHUNK = PER_W // CHUNK
SUB = CHUNK // L

TC_R, TC_C = 784, 256


def _sc_gather(img, ya, xa, yb, xb):
    mesh = plsc.VectorSubcoreMesh(
        core_axis_name="c", subcore_axis_name="s", num_cores=NC, num_subcores=NS
    )

    def body(img_hbm, ya_hbm, xa_hbm, yb_hbm, xb_hbm, d_hbm,
             ya_v, xa_v, yb_v, xb_v, idxa_v, idxb_v, za_v, zb_v, sem, wsem):
        wid = lax.axis_index("s") * NC + lax.axis_index("c")
        batch = lax.shift_right_logical(wid, 3)
        boff = batch * (H * W)
        base = batch * PPAD + (wid & (WPB - 1)) * PER_W

        cps = [
            pltpu.async_copy(src.at[pl.ds(base, PER_W)], dst, sem)
            for src, dst in ((ya_hbm, ya_v), (xa_hbm, xa_v),
                             (yb_hbm, yb_v), (xb_hbm, xb_v))
        ]
        for cp in cps:
            cp.wait()

        def chunk_body(c, carry):
            for u in range(SUB):
                off = c * CHUNK + u * L
                idxa_v[pl.ds(off, L)] = boff + ya_v[pl.ds(off, L)] * W + xa_v[pl.ds(off, L)]
                idxb_v[pl.ds(off, L)] = boff + yb_v[pl.ds(off, L)] * W + xb_v[pl.ds(off, L)]
            co = c * CHUNK
            pltpu.async_copy(img_hbm.at[idxa_v.at[pl.ds(co, CHUNK)]],
                             za_v.at[pl.ds(co, CHUNK)], sem)
            pltpu.async_copy(img_hbm.at[idxb_v.at[pl.ds(co, CHUNK)]],
                             zb_v.at[pl.ds(co, CHUNK)], sem)
            return carry

        lax.fori_loop(0, NCHUNK, chunk_body, 0)

        pltpu.make_async_copy(img_hbm.at[pl.ds(0, PER_W)], za_v, sem).wait()
        pltpu.make_async_copy(img_hbm.at[pl.ds(0, PER_W)], zb_v, sem).wait()

        def diff_body(j, carry):
            off = j * L
            za_v[pl.ds(off, L)] = za_v[pl.ds(off, L)] - zb_v[pl.ds(off, L)]
            return carry

        lax.fori_loop(0, PER_W // L, diff_body, 0)
        pltpu.async_copy(za_v, d_hbm.at[pl.ds(base, PER_W)], wsem).wait()

    f = pl.kernel(
        body,
        out_type=jax.ShapeDtypeStruct((NPAD,), jnp.float32),
        mesh=mesh,
        scratch_types=[
            pltpu.VMEM((PER_W,), jnp.int32),
            pltpu.VMEM((PER_W,), jnp.int32),
            pltpu.VMEM((PER_W,), jnp.int32),
            pltpu.VMEM((PER_W,), jnp.int32),
            pltpu.VMEM((PER_W,), jnp.int32),
            pltpu.VMEM((PER_W,), jnp.int32),
            pltpu.VMEM((PER_W,), jnp.float32),
            pltpu.VMEM((PER_W,), jnp.float32),
            pltpu.SemaphoreType.DMA,
            pltpu.SemaphoreType.DMA,
        ],
    )
    return f(img, ya, xa, yb, xb)


def _tc_loss_body(d_ref, o_ref, out_ref):
    d = d_ref[...]
    o = o_ref[...]
    mask = jnp.abs(o)
    t = jnp.minimum(o * d, MARGIN)
    loss = mask * jnp.log(1.0 + jnp.exp(-t)) \
        + (1.0 - mask) * jnp.maximum(d * d, MARGIN * MARGIN)
    pos = (lax.broadcasted_iota(jnp.int32, (TC_R, TC_C), 0) * TC_C
           + lax.broadcasted_iota(jnp.int32, (TC_R, TC_C), 1))
    loss = jnp.where(pos % PPAD < P, loss, 0.0)
    out_ref[0, 0] = jnp.sum(loss) / NPTS


def _tc_loss(d, o):
    return pl.pallas_call(
        _tc_loss_body,
        out_shape=jax.ShapeDtypeStruct((1, 1), jnp.float32),
        out_specs=pl.BlockSpec(memory_space=pltpu.SMEM),
    )(d, o)


def _pad_rows_i32(a):
    return jnp.pad(a.astype(jnp.int32), ((0, 0), (0, PPAD - P))).reshape(-1)


def kernel(input, x_A, y_A, x_B, y_B, ordinal):
    img = input.reshape(B * H * W)
    ya = _pad_rows_i32(y_A)
    xa = _pad_rows_i32(x_A)
    yb = _pad_rows_i32(y_B)
    xb = _pad_rows_i32(x_B)
    o = jnp.pad(ordinal.astype(jnp.float32), ((0, 0), (0, PPAD - P))).reshape(-1)
    d = _sc_gather(img, ya, xa, yb, xb)
    out = _tc_loss(d.reshape(TC_R, TC_C), o.reshape(TC_R, TC_C))
    return out[0, 0]


# speedup vs baseline: 1.2213x; 1.0658x over previous
import jax
import jax.numpy as jnp
from jax import lax
from jax.experimental import pallas as pl
from jax.experimental.pallas import tpu as pltpu
from jax.experimental.pallas import tpu_sc as plsc

B, H, W, P = 4, 512, 512, 50000
MARGIN = 1.0
NPTS = B * P
NC, NS, L = 2, 16, 16
NW = NC * NS
WPB = NW // B
PPAD = 50176
NPAD = B * PPAD
PADC = float(NPAD - NPTS)
PER_W = PPAD // WPB
CHUNK = 128
NCHUNK = PER_W // CHUNK
SUB = CHUNK // L
QCH = (13, 12, 12, 12)
QST = (0, 13, 25, 37)

LN2 = 0.6931471805599453
_LOG2C = (7.283239262169318e-06, 1.4423285361122946, -0.7164483783618765,
          0.45208220030532664, -0.26961100983901826, 0.11592938544152971,
          -0.02429299844067783)


def _softplus(t):
    v = 1.0 + jnp.exp(-t)
    bits = plsc.bitcast(v, jnp.int32)
    e = lax.shift_right_logical(bits, 23) - 127
    m = plsc.bitcast(
        lax.bitwise_or(lax.bitwise_and(bits, 0x7FFFFF), 0x3F800000),
        jnp.float32)
    f = m - 1.0
    p = jnp.float32(_LOG2C[6])
    for co in _LOG2C[5::-1]:
        p = p * f + jnp.float32(co)
    return LN2 * (e.astype(jnp.float32) + p)


def _sc_loss(img, ya, xa, yb, xb, o):
    mesh = plsc.VectorSubcoreMesh(
        core_axis_name="c", subcore_axis_name="s", num_cores=NC, num_subcores=NS
    )

    def body(img_hbm, ya_hbm, xa_hbm, yb_hbm, xb_hbm, o_hbm, out_hbm,
             ya_v, xa_v, yb_v, xb_v, o_v, idxa_v, idxb_v, za_v, zb_v, acc_v,
             sem0, sem1, sem2, sem3):
        sems = (sem0, sem1, sem2, sem3)
        wid = lax.axis_index("s") * NC + lax.axis_index("c")
        batch = lax.shift_right_logical(wid, 3)
        boff = batch * (H * W)
        base = batch * PPAD + (wid & (WPB - 1)) * PER_W

        cps = [
            pltpu.async_copy(src.at[pl.ds(base, PER_W)], dst, sem0)
            for src, dst in ((ya_hbm, ya_v), (xa_hbm, xa_v),
                             (yb_hbm, yb_v), (xb_hbm, xb_v), (o_hbm, o_v))
        ]
        for cp in cps:
            cp.wait()

        def make_issue(q):
            sem = sems[q]

            def chunk_body(c, carry):
                for u in range(SUB):
                    off = c * CHUNK + u * L
                    idxa_v[pl.ds(off, L)] = (boff + ya_v[pl.ds(off, L)] * W
                                             + xa_v[pl.ds(off, L)])
                    idxb_v[pl.ds(off, L)] = (boff + yb_v[pl.ds(off, L)] * W
                                             + xb_v[pl.ds(off, L)])
                co = c * CHUNK
                pltpu.async_copy(img_hbm.at[idxa_v.at[pl.ds(co, CHUNK)]],
                                 za_v.at[pl.ds(co, CHUNK)], sem)
                pltpu.async_copy(img_hbm.at[idxb_v.at[pl.ds(co, CHUNK)]],
                                 zb_v.at[pl.ds(co, CHUNK)], sem)
                return carry

            lax.fori_loop(QST[q], QST[q] + QCH[q], chunk_body, 0)

        def quarter_wait(q):
            n = QCH[q] * CHUNK * 2
            pltpu.make_async_copy(img_hbm.at[pl.ds(0, n)],
                                  za_v.at[pl.ds(0, n)], sems[q]).wait()

        def make_loss(q, acc):
            def chunk_body(c, acc):
                for u in range(SUB):
                    off = c * CHUNK + u * L
                    d = za_v[pl.ds(off, L)] - zb_v[pl.ds(off, L)]
                    o_s = o_v[pl.ds(off, L)]
                    mask = jnp.abs(o_s)
                    t = jnp.minimum(o_s * d, MARGIN)
                    sp = _softplus(t)
                    q_br = jnp.maximum(d * d, MARGIN * MARGIN)
                    acc = acc + (mask * sp + (1.0 - mask) * q_br)
                return acc

            return lax.fori_loop(QST[q], QST[q] + QCH[q], chunk_body, acc)

        for q in range(4):
            make_issue(q)
        acc = jnp.zeros((L,), jnp.float32)
        for q in range(4):
            quarter_wait(q)
            acc = make_loss(q, acc)

        acc_v[...] = acc
        pltpu.sync_copy(acc_v, out_hbm.at[pl.ds(wid * L, L)])

    f = pl.kernel(
        body,
        out_type=jax.ShapeDtypeStruct((NW * L,), jnp.float32),
        mesh=mesh,
        compiler_params=pltpu.CompilerParams(needs_layout_passes=False),
        scratch_types=[
            pltpu.VMEM((PER_W,), jnp.int32),
            pltpu.VMEM((PER_W,), jnp.int32),
            pltpu.VMEM((PER_W,), jnp.int32),
            pltpu.VMEM((PER_W,), jnp.int32),
            pltpu.VMEM((PER_W,), jnp.float32),
            pltpu.VMEM((PER_W,), jnp.int32),
            pltpu.VMEM((PER_W,), jnp.int32),
            pltpu.VMEM((PER_W,), jnp.float32),
            pltpu.VMEM((PER_W,), jnp.float32),
            pltpu.VMEM((L,), jnp.float32),
            pltpu.SemaphoreType.DMA,
            pltpu.SemaphoreType.DMA,
            pltpu.SemaphoreType.DMA,
            pltpu.SemaphoreType.DMA,
        ],
    )
    return f(img, ya, xa, yb, xb, o)


def _tc_fin_body(p_ref, out_ref):
    out_ref[0, 0] = (jnp.sum(p_ref[...]) - PADC) / NPTS


def _tc_fin(partials):
    return pl.pallas_call(
        _tc_fin_body,
        out_shape=jax.ShapeDtypeStruct((1, 1), jnp.float32),
        out_specs=pl.BlockSpec(memory_space=pltpu.SMEM),
    )(partials)


def _pad_rows_i32(a):
    return jnp.pad(a.astype(jnp.int32), ((0, 0), (0, PPAD - P))).reshape(-1)


def kernel(input, x_A, y_A, x_B, y_B, ordinal):
    img = input.reshape(B * H * W)
    ya = _pad_rows_i32(y_A)
    xa = _pad_rows_i32(x_A)
    yb = _pad_rows_i32(y_B)
    xb = _pad_rows_i32(x_B)
    o = jnp.pad(ordinal.astype(jnp.float32), ((0, 0), (0, PPAD - P))).reshape(-1)
    partials = _sc_loss(img, ya, xa, yb, xb, o)
    out = _tc_fin(partials.reshape(4, 128))
    return out[0, 0]
